# baseline (speedup 1.0000x reference)
import functools

import jax
import jax.numpy as jnp
from jax import lax
from jax.experimental import pallas as pl
from jax.experimental.pallas import tpu as pltpu
from jax.experimental.pallas import tpu_sc as plsc

NC, NS = 2, 16
NW = NC * NS
G = 128


def _make_sc_gather(V, D, N, per_w, gpc_rows, n_chunks):
    ch = gpc_rows * G
    assert n_chunks % 2 == 0

    mesh = plsc.VectorSubcoreMesh(core_axis_name="c", subcore_axis_name="s")

    rows_per_w = per_w // G

    @functools.partial(
        pl.kernel,
        out_type=jax.ShapeDtypeStruct((N, D), jnp.float32),
        mesh=mesh,
        scratch_types=[
            pltpu.VMEM((rows_per_w, G), jnp.int32),
            pltpu.VMEM((ch, D), jnp.float32),
            pltpu.VMEM((ch, D), jnp.float32),
            pltpu.SemaphoreType.DMA,
            pltpu.SemaphoreType.DMA,
        ],
    )
    def sc_gather(table_hbm, idx_hbm, out_hbm, idx_v, rows_a, rows_b,
                  gsem, osem):
        wid = lax.axis_index("s") * NC + lax.axis_index("c")
        out_row0 = wid * per_w

        pltpu.sync_copy(idx_hbm.at[wid], idx_v)

        def gather_chunk(ci, buf):
            r0 = ci * gpc_rows
            copies = []
            for j in range(gpc_rows):
                copies.append(pltpu.async_copy(
                    table_hbm.at[idx_v.at[r0 + j]],
                    buf.at[pl.ds(j * G, G)],
                    gsem))
            for cp in copies:
                cp.wait()

        def pair(p, carry):
            c0 = 2 * p
            gather_chunk(c0, rows_a)
            copy_a = pltpu.async_copy(
                rows_a, out_hbm.at[pl.ds(out_row0 + c0 * ch, ch)], osem)
            gather_chunk(c0 + 1, rows_b)
            copy_a.wait()
            pltpu.async_copy(
                rows_b, out_hbm.at[pl.ds(out_row0 + (c0 + 1) * ch, ch)],
                osem).wait()
            return carry

        lax.fori_loop(0, n_chunks // 2, pair, 0)

    return sc_gather


def _score_body(d_ref, q_ref, w_ref, b_ref, out_ref):
    BB = out_ref.shape[0]
    D = d_ref.shape[1]
    LD = d_ref.shape[0] // BB
    LQ = q_ref.shape[0] // BB
    d = d_ref[...]
    q = q_ref[...]
    qn = q / (jnp.sqrt(jnp.sum(q * q, axis=1, keepdims=True)) + 1e-10)
    sqd = d * d
    ones = jnp.ones((1, D), jnp.float32)
    w = w_ref[0, 0]
    b = b_ref[0, 0]
    for bi in range(BB):
        qb = lax.slice(qn, (bi * LQ, 0), ((bi + 1) * LQ, D))
        db = lax.slice(d, (bi * LD, 0), ((bi + 1) * LD, D))
        sb = lax.slice(sqd, (bi * LD, 0), ((bi + 1) * LD, D))
        raw = lax.dot_general(qb, db, (((1,), (1,)), ((), ())),
                              preferred_element_type=jnp.float32)
        nd = lax.dot_general(ones, sb, (((1,), (1,)), ((), ())),
                             preferred_element_type=jnp.float32)
        sim = raw * (1.0 / (jnp.sqrt(nd) + 1e-10))
        mx = jnp.max(sim, axis=1)
        pooled = jnp.sum(mx)
        out_ref[bi, :] = jnp.full((D,), pooled * w + b, dtype=jnp.float32)


NUM_SLABS = 1
BB = 64
GPC = 2


def kernel(query_tokens, document_tokens, embedding_table, score_w, score_b):
    B, LQ = query_tokens.shape
    _, LD = document_tokens.shape
    V, D = embedding_table.shape
    table = embedding_table.astype(jnp.float32)
    w2d = jnp.reshape(score_w, (1, 1)).astype(jnp.float32)
    b2d = jnp.reshape(score_b, (1, 1)).astype(jnp.float32)
    qt = query_tokens.astype(jnp.int32)
    dt = document_tokens.astype(jnp.int32)

    S = NUM_SLABS
    Bs = B // S
    assert Bs * S == B and Bs % BB == 0
    ND = Bs * LD
    n_s = ND + Bs * LQ
    grain = NW * G * (2 * GPC)
    n_pad = ((n_s + grain - 1) // grain) * grain
    per_w = n_pad // NW
    rows_per_w = per_w // G
    gpc_rows = GPC
    n_chunks = rows_per_w // gpc_rows

    sc_gather = _make_sc_gather(V, D, n_pad, per_w, gpc_rows, n_chunks)
    q_blk0 = ND // (BB * LQ)

    outs = []
    for s in range(S):
        dts = lax.slice(dt, (s * Bs, 0), ((s + 1) * Bs, LD))
        qts = lax.slice(qt, (s * Bs, 0), ((s + 1) * Bs, LQ))
        idx = jnp.concatenate([
            dts.reshape(-1), qts.reshape(-1),
            jnp.zeros((n_pad - n_s,), jnp.int32)])
        gathered = sc_gather(table, idx.reshape(NW, rows_per_w, G))
        out = pl.pallas_call(
            _score_body,
            grid=(Bs // BB,),
            in_specs=[
                pl.BlockSpec((BB * LD, D), lambda i: (i, 0)),
                pl.BlockSpec((BB * LQ, D), lambda i: (i + q_blk0, 0)),
                pl.BlockSpec(memory_space=pltpu.SMEM),
                pl.BlockSpec(memory_space=pltpu.SMEM),
            ],
            out_specs=pl.BlockSpec((BB, D), lambda i: (i, 0)),
            out_shape=jax.ShapeDtypeStruct((Bs, D), jnp.float32),
            compiler_params=pltpu.CompilerParams(
                dimension_semantics=("parallel",)),
        )(gathered, gathered, w2d, b2d)
        outs.append(out[:, 0])
    return jnp.concatenate(outs)


# speedup vs baseline: 1.0020x; 1.0020x over previous
import functools

import jax
import jax.numpy as jnp
from jax import lax
from jax.experimental import pallas as pl
from jax.experimental.pallas import tpu as pltpu
from jax.experimental.pallas import tpu_sc as plsc

NC, NS = 2, 16
NW = NC * NS
G = 128


def _make_sc_gather(V, D, N, per_w, gpc_rows, n_chunks):
    ch = gpc_rows * G

    mesh = plsc.VectorSubcoreMesh(core_axis_name="c", subcore_axis_name="s")

    rows_per_w = per_w // G

    @functools.partial(
        pl.kernel,
        out_type=jax.ShapeDtypeStruct((N, D), jnp.float32),
        mesh=mesh,
        scratch_types=[
            pltpu.VMEM((rows_per_w, G), jnp.int32),
            pltpu.VMEM((ch, D), jnp.float32),
            pltpu.SemaphoreType.DMA,
        ],
    )
    def sc_gather(table_hbm, idx_hbm, out_hbm, idx_v, rows_v, gsem):
        wid = lax.axis_index("s") * NC + lax.axis_index("c")
        out_row0 = wid * per_w

        pltpu.sync_copy(idx_hbm.at[wid], idx_v)

        def chunk(ci, carry):
            r0 = ci * gpc_rows
            copies = []
            for j in range(gpc_rows):
                copies.append(pltpu.async_copy(
                    table_hbm.at[idx_v.at[r0 + j]],
                    rows_v.at[pl.ds(j * G, G)],
                    gsem))
            for cp in copies:
                cp.wait()
            pltpu.sync_copy(rows_v, out_hbm.at[pl.ds(out_row0 + ci * ch, ch)])
            return carry

        lax.fori_loop(0, n_chunks, chunk, 0)

    return sc_gather


def _score_body(d_ref, q_ref, w_ref, b_ref, out_ref):
    BB = out_ref.shape[0]
    D = d_ref.shape[1]
    LD = d_ref.shape[0] // BB
    LQ = q_ref.shape[0] // BB
    d = d_ref[...]
    q = q_ref[...]
    qn = q / (jnp.sqrt(jnp.sum(q * q, axis=1, keepdims=True)) + 1e-10)
    sqd = d * d
    ones = jnp.ones((1, D), jnp.float32)
    w = w_ref[0, 0]
    b = b_ref[0, 0]
    for bi in range(BB):
        qb = lax.slice(qn, (bi * LQ, 0), ((bi + 1) * LQ, D))
        db = lax.slice(d, (bi * LD, 0), ((bi + 1) * LD, D))
        sb = lax.slice(sqd, (bi * LD, 0), ((bi + 1) * LD, D))
        raw = lax.dot_general(qb, db, (((1,), (1,)), ((), ())),
                              preferred_element_type=jnp.float32)
        nd = lax.dot_general(ones, sb, (((1,), (1,)), ((), ())),
                             preferred_element_type=jnp.float32)
        sim = raw * (1.0 / (jnp.sqrt(nd) + 1e-10))
        mx = jnp.max(sim, axis=1)
        pooled = jnp.sum(mx)
        out_ref[bi, :] = jnp.full((D,), pooled * w + b, dtype=jnp.float32)


NUM_SLABS = 1
BB = 64
GPC = 7


def kernel(query_tokens, document_tokens, embedding_table, score_w, score_b):
    B, LQ = query_tokens.shape
    _, LD = document_tokens.shape
    V, D = embedding_table.shape
    table = embedding_table.astype(jnp.float32)
    w2d = jnp.reshape(score_w, (1, 1)).astype(jnp.float32)
    b2d = jnp.reshape(score_b, (1, 1)).astype(jnp.float32)
    qt = query_tokens.astype(jnp.int32)
    dt = document_tokens.astype(jnp.int32)

    S = NUM_SLABS
    Bs = B // S
    assert Bs * S == B and Bs % BB == 0
    ND = Bs * LD
    n_s = ND + Bs * LQ
    grain = NW * G * GPC
    n_pad = ((n_s + grain - 1) // grain) * grain
    per_w = n_pad // NW
    rows_per_w = per_w // G
    gpc_rows = GPC
    n_chunks = rows_per_w // gpc_rows

    sc_gather = _make_sc_gather(V, D, n_pad, per_w, gpc_rows, n_chunks)
    q_blk0 = ND // (BB * LQ)

    outs = []
    for s in range(S):
        dts = lax.slice(dt, (s * Bs, 0), ((s + 1) * Bs, LD))
        qts = lax.slice(qt, (s * Bs, 0), ((s + 1) * Bs, LQ))
        idx = jnp.concatenate([
            dts.reshape(-1), qts.reshape(-1),
            jnp.zeros((n_pad - n_s,), jnp.int32)])
        gathered = sc_gather(table, idx.reshape(NW, rows_per_w, G))
        out = pl.pallas_call(
            _score_body,
            grid=(Bs // BB,),
            in_specs=[
                pl.BlockSpec((BB * LD, D), lambda i: (i, 0)),
                pl.BlockSpec((BB * LQ, D), lambda i: (i + q_blk0, 0)),
                pl.BlockSpec(memory_space=pltpu.SMEM),
                pl.BlockSpec(memory_space=pltpu.SMEM),
            ],
            out_specs=pl.BlockSpec((BB, D), lambda i: (i, 0)),
            out_shape=jax.ShapeDtypeStruct((Bs, D), jnp.float32),
            compiler_params=pltpu.CompilerParams(
                dimension_semantics=("parallel",)),
        )(gathered, gathered, w2d, b2d)
        outs.append(out[:, 0])
    return jnp.concatenate(outs)


# speedup vs baseline: 2.5319x; 2.5269x over previous
import functools

import jax
import jax.numpy as jnp
from jax import lax
from jax.experimental import pallas as pl
from jax.experimental.pallas import tpu as pltpu
from jax.experimental.pallas import tpu_sc as plsc

NC, NS = 2, 16
NW = NC * NS
G = 128


def _make_sc_gather(V, D, N, per_w, gpc_rows, n_chunks):
    ch = gpc_rows * G

    mesh = plsc.VectorSubcoreMesh(core_axis_name="c", subcore_axis_name="s")

    rows_per_w = per_w // G

    @functools.partial(
        pl.kernel,
        out_type=jax.ShapeDtypeStruct((N, D), jnp.float32),
        mesh=mesh,
        scratch_types=[
            pltpu.VMEM((rows_per_w, G), jnp.int32),
            pltpu.VMEM((ch, D), jnp.float32),
            pltpu.SemaphoreType.DMA,
        ],
    )
    def sc_gather(table_hbm, idx_hbm, out_hbm, idx_v, rows_v, gsem):
        wid = lax.axis_index("s") * NC + lax.axis_index("c")
        out_row0 = wid * per_w

        pltpu.sync_copy(idx_hbm.at[wid], idx_v)

        def chunk(ci, carry):
            r0 = ci * gpc_rows
            copies = []
            for j in range(gpc_rows):
                copies.append(pltpu.async_copy(
                    table_hbm.at[idx_v.at[r0 + j]],
                    rows_v.at[pl.ds(j * G, G)],
                    gsem))
            for cp in copies:
                cp.wait()
            pltpu.sync_copy(rows_v, out_hbm.at[pl.ds(out_row0 + ci * ch, ch)])
            return carry

        lax.fori_loop(0, n_chunks, chunk, 0)

    return sc_gather


def _score_body(d_ref, q_ref, w_ref, b_ref, out_ref):
    BB = out_ref.shape[0]
    D = d_ref.shape[1]
    LD = d_ref.shape[0] // BB
    LQ = q_ref.shape[0] // BB
    d = d_ref[...]
    q = q_ref[...]
    qn = q / (jnp.sqrt(jnp.sum(q * q, axis=1, keepdims=True)) + 1e-10)
    sqd = d * d
    ones = jnp.ones((1, D), jnp.float32)
    w = w_ref[0, 0]
    b = b_ref[0, 0]
    for bi in range(BB):
        qb = lax.slice(qn, (bi * LQ, 0), ((bi + 1) * LQ, D))
        db = lax.slice(d, (bi * LD, 0), ((bi + 1) * LD, D))
        sb = lax.slice(sqd, (bi * LD, 0), ((bi + 1) * LD, D))
        raw = lax.dot_general(qb, db, (((1,), (1,)), ((), ())),
                              preferred_element_type=jnp.float32)
        nd = lax.dot_general(ones, sb, (((1,), (1,)), ((), ())),
                             preferred_element_type=jnp.float32)
        sim = raw * (1.0 / (jnp.sqrt(nd) + 1e-10))
        mx = jnp.max(sim, axis=1)
        pooled = jnp.sum(mx)
        out_ref[bi, :] = jnp.full((D,), pooled * w + b, dtype=jnp.float32)


NUM_SLABS = 1
BB = 64
GPC = 7


def kernel(query_tokens, document_tokens, embedding_table, score_w, score_b):
    B, LQ = query_tokens.shape
    _, LD = document_tokens.shape
    V, D = embedding_table.shape
    table = embedding_table.astype(jnp.float32)
    w2d = jnp.reshape(score_w, (1, 1)).astype(jnp.float32)
    b2d = jnp.reshape(score_b, (1, 1)).astype(jnp.float32)
    qt = query_tokens.astype(jnp.int32)
    dt = document_tokens.astype(jnp.int32)

    S = NUM_SLABS
    Bs = B // S
    assert Bs * S == B and Bs % BB == 0
    ND = Bs * LD
    n_s = ND + Bs * LQ
    grain = NW * G * GPC
    n_pad = ((n_s + grain - 1) // grain) * grain
    per_w = n_pad // NW
    rows_per_w = per_w // G
    gpc_rows = GPC
    n_chunks = rows_per_w // gpc_rows

    sc_gather = _make_sc_gather(V, D, n_pad, per_w, gpc_rows, n_chunks)
    q_blk0 = ND // (BB * LQ)

    outs = []
    for s in range(S):
        dts = lax.slice(dt, (s * Bs, 0), ((s + 1) * Bs, LD))
        qts = lax.slice(qt, (s * Bs, 0), ((s + 1) * Bs, LQ))
        idx = jnp.concatenate([
            dts.reshape(-1), qts.reshape(-1),
            jnp.arange(n_pad - n_s, dtype=jnp.int32)])
        gathered = sc_gather(table, idx.reshape(NW, rows_per_w, G))
        out = pl.pallas_call(
            _score_body,
            grid=(Bs // BB,),
            in_specs=[
                pl.BlockSpec((BB * LD, D), lambda i: (i, 0)),
                pl.BlockSpec((BB * LQ, D), lambda i: (i + q_blk0, 0)),
                pl.BlockSpec(memory_space=pltpu.SMEM),
                pl.BlockSpec(memory_space=pltpu.SMEM),
            ],
            out_specs=pl.BlockSpec((BB, D), lambda i: (i, 0)),
            out_shape=jax.ShapeDtypeStruct((Bs, D), jnp.float32),
            compiler_params=pltpu.CompilerParams(
                dimension_semantics=("parallel",)),
        )(gathered, gathered, w2d, b2d)
        outs.append(out[:, 0])
    return jnp.concatenate(outs)


# speedup vs baseline: 2.5513x; 1.0077x over previous
import functools

import jax
import jax.numpy as jnp
from jax import lax
from jax.experimental import pallas as pl
from jax.experimental.pallas import tpu as pltpu
from jax.experimental.pallas import tpu_sc as plsc

NC, NS = 2, 16
NW = NC * NS
G = 128


def _make_sc_gather(V, D, N, per_w, gpc_rows, n_chunks):
    ch = gpc_rows * G

    mesh = plsc.VectorSubcoreMesh(core_axis_name="c", subcore_axis_name="s")

    rows_per_w = per_w // G

    @functools.partial(
        pl.kernel,
        out_type=jax.ShapeDtypeStruct((N, D), jnp.float32),
        mesh=mesh,
        scratch_types=[
            pltpu.VMEM((rows_per_w, G), jnp.int32),
            pltpu.VMEM((ch, D), jnp.float32),
            pltpu.SemaphoreType.DMA,
        ],
    )
    def sc_gather(table_hbm, idx_hbm, out_hbm, idx_v, rows_v, gsem):
        wid = lax.axis_index("s") * NC + lax.axis_index("c")
        out_row0 = wid * per_w

        pltpu.sync_copy(idx_hbm.at[wid], idx_v)

        def chunk(ci, carry):
            r0 = ci * gpc_rows
            copies = []
            for j in range(gpc_rows):
                copies.append(pltpu.async_copy(
                    table_hbm.at[idx_v.at[r0 + j]],
                    rows_v.at[pl.ds(j * G, G)],
                    gsem))
            for cp in copies:
                cp.wait()
            pltpu.sync_copy(rows_v, out_hbm.at[pl.ds(out_row0 + ci * ch, ch)])
            return carry

        lax.fori_loop(0, n_chunks, chunk, 0)

    return sc_gather


def _score_body(d_ref, q_ref, w_ref, b_ref, out_ref):
    BB = out_ref.shape[0]
    D = d_ref.shape[1]
    LD = d_ref.shape[0] // BB
    LQ = q_ref.shape[0] // BB
    d = d_ref[...]
    q = q_ref[...]
    qn = q / (jnp.sqrt(jnp.sum(q * q, axis=1, keepdims=True)) + 1e-10)
    sqd = d * d
    ones = jnp.ones((1, D), jnp.float32)
    w = w_ref[0, 0]
    b = b_ref[0, 0]
    for bi in range(BB):
        qb = lax.slice(qn, (bi * LQ, 0), ((bi + 1) * LQ, D))
        db = lax.slice(d, (bi * LD, 0), ((bi + 1) * LD, D))
        sb = lax.slice(sqd, (bi * LD, 0), ((bi + 1) * LD, D))
        raw = lax.dot_general(qb, db, (((1,), (1,)), ((), ())),
                              preferred_element_type=jnp.float32)
        nd = lax.dot_general(ones, sb, (((1,), (1,)), ((), ())),
                             preferred_element_type=jnp.float32)
        sim = raw * (1.0 / (jnp.sqrt(nd) + 1e-10))
        mx = jnp.max(sim, axis=1)
        pooled = jnp.sum(mx)
        out_ref[bi, :] = jnp.full((D,), pooled * w + b, dtype=jnp.float32)


NUM_SLABS = 1
BB = 64
GPC = 5


def kernel(query_tokens, document_tokens, embedding_table, score_w, score_b):
    B, LQ = query_tokens.shape
    _, LD = document_tokens.shape
    V, D = embedding_table.shape
    table = embedding_table.astype(jnp.float32)
    w2d = jnp.reshape(score_w, (1, 1)).astype(jnp.float32)
    b2d = jnp.reshape(score_b, (1, 1)).astype(jnp.float32)
    qt = query_tokens.astype(jnp.int32)
    dt = document_tokens.astype(jnp.int32)

    S = NUM_SLABS
    Bs = B // S
    assert Bs * S == B and Bs % BB == 0
    ND = Bs * LD
    n_s = ND + Bs * LQ
    grain = NW * G * GPC
    n_pad = ((n_s + grain - 1) // grain) * grain
    per_w = n_pad // NW
    rows_per_w = per_w // G
    gpc_rows = GPC
    n_chunks = rows_per_w // gpc_rows

    sc_gather = _make_sc_gather(V, D, n_pad, per_w, gpc_rows, n_chunks)
    q_blk0 = ND // (BB * LQ)

    outs = []
    for s in range(S):
        dts = lax.slice(dt, (s * Bs, 0), ((s + 1) * Bs, LD))
        qts = lax.slice(qt, (s * Bs, 0), ((s + 1) * Bs, LQ))
        idx = jnp.concatenate([
            dts.reshape(-1), qts.reshape(-1),
            jnp.arange(n_pad - n_s, dtype=jnp.int32)])
        gathered = sc_gather(table, idx.reshape(NW, rows_per_w, G))
        out = pl.pallas_call(
            _score_body,
            grid=(Bs // BB,),
            in_specs=[
                pl.BlockSpec((BB * LD, D), lambda i: (i, 0)),
                pl.BlockSpec((BB * LQ, D), lambda i: (i + q_blk0, 0)),
                pl.BlockSpec(memory_space=pltpu.SMEM),
                pl.BlockSpec(memory_space=pltpu.SMEM),
            ],
            out_specs=pl.BlockSpec((BB, D), lambda i: (i, 0)),
            out_shape=jax.ShapeDtypeStruct((Bs, D), jnp.float32),
            compiler_params=pltpu.CompilerParams(
                dimension_semantics=("parallel",)),
        )(gathered, gathered, w2d, b2d)
        outs.append(out[:, 0])
    return jnp.concatenate(outs)
